# baseline (speedup 1.0000x reference)
import jax
import jax.numpy as jnp
from jax import lax
from jax.experimental import pallas as pl
from jax.experimental.pallas import tpu as pltpu
from jax.experimental.pallas import tpu_sc as plsc

B, S, D = 16, 2048, 1024
L = 16
K = 32
SEG = K + 9
NB = 3
HALF = S // 2
G = HALF // K


def _body(bank, refresh, out, buf, bnd, sem_g, sem_b, sem_s):
    cid = lax.axis_index("c")
    sid = lax.axis_index("s")
    wid = sid * 2 + cid
    b = wid // 2
    h = wid % 2
    base = h * HALF

    @pl.when(h == 1)
    def _():
        cp = pltpu.make_async_copy(refresh.at[b], bnd, sem_b)
        cp.start()
        cp.wait()

    def is_last_h1(g):
        return jnp.logical_and(g == G - 1, h == 1)

    def gather_start(g):
        seg = g % NB
        off = base + g * K - jnp.where(is_last_h1(g), 8, 0)
        pltpu.make_async_copy(
            bank.at[b, pl.ds(off, K + 8)],
            buf.at[pl.ds(seg * SEG, K + 8)],
            sem_g,
        ).start()

    def gather_wait():
        pltpu.make_async_copy(
            bank.at[b, pl.ds(base, K + 8)], buf.at[pl.ds(0, K + 8)], sem_g
        ).wait()

    def scatter_start(g):
        seg = g % NB
        src_off = seg * SEG + 1 + jnp.where(is_last_h1(g), 8, 0)
        pltpu.make_async_copy(
            buf.at[pl.ds(src_off, K)],
            out.at[b, pl.ds(base + g * K, K)],
            sem_s,
        ).start()

    def scatter_wait():
        pltpu.make_async_copy(
            buf.at[pl.ds(0, K)], out.at[b, pl.ds(base, K)], sem_s
        ).wait()

    gather_start(0)

    def chunk(g, carry):
        seg = g % NB

        @pl.when(g + 1 < G)
        def _():
            @pl.when(g + 1 >= NB)
            def _():
                scatter_wait()

            gather_start(g + 1)

        gather_wait()

        sb = seg * SEG

        @pl.when(jnp.logical_and(g == 0, h == 0))
        def _():
            for c in range(D // L):
                sl = pl.ds(c * L, L)
                buf[sb + 1, sl] = 0.5 * (buf[sb, sl] + buf[sb + 1, sl])

        @pl.when(is_last_h1(g))
        def _():
            for c in range(D // L):
                sl = pl.ds(c * L, L)
                buf[sb + K + 8, sl] = bnd[0, sl]

        scatter_start(g)
        return carry

    lax.fori_loop(0, G, chunk, 0)
    for _ in range(min(NB, G)):
        scatter_wait()


@jax.jit
def _shift(bank_states, refresh_states):
    mesh = plsc.VectorSubcoreMesh(core_axis_name="c", subcore_axis_name="s")
    return pl.kernel(
        _body,
        mesh=mesh,
        out_type=jax.ShapeDtypeStruct((B, S, D), jnp.float32),
        compiler_params=pltpu.CompilerParams(use_tc_tiling_on_sc=False),
        scratch_types=[
            pltpu.VMEM((NB * SEG, D), jnp.float32),
            pltpu.VMEM((1, D), jnp.float32),
            pltpu.SemaphoreType.DMA,
            pltpu.SemaphoreType.DMA,
            pltpu.SemaphoreType.DMA,
        ],
    )(bank_states, refresh_states)


def kernel(bank_states, refresh_states):
    return _shift(bank_states, refresh_states)


# speedup vs baseline: 1.2112x; 1.2112x over previous
import jax
import jax.numpy as jnp
from jax import lax
from jax.experimental import pallas as pl
from jax.experimental.pallas import tpu as pltpu
from jax.experimental.pallas import tpu_sc as plsc

B, S, D = 16, 2048, 1024
L = 16
K = 32
NBI = 2
HALF = S // 2
G = HALF // K


def _copy_row(dst_ref, dst_row, src_ref, src_row):
    for c in range(D // L):
        sl = pl.ds(c * L, L)
        dst_ref[dst_row, sl] = src_ref[src_row, sl]


def _body(bank, refresh, out, ibuf, obuf, bnd, sem_g, sem_b, sem_s):
    cid = lax.axis_index("c")
    sid = lax.axis_index("s")
    wid = sid * 2 + cid
    b = wid // 2
    h = wid % 2
    base = h * HALF

    @pl.when(h == 0)
    def _():
        pltpu.make_async_copy(bank.at[b, pl.ds(HALF, 1)], bnd, sem_b).start()

    @pl.when(h == 1)
    def _():
        pltpu.make_async_copy(refresh.at[b], bnd, sem_b).start()

    def gather_start(g):
        pltpu.make_async_copy(
            bank.at[b, pl.ds(base + g * K, K)],
            ibuf.at[pl.ds((g % NBI) * K, K)],
            sem_g,
        ).start()

    def gather_wait():
        pltpu.make_async_copy(
            bank.at[b, pl.ds(base, K)], ibuf.at[pl.ds(0, K)], sem_g
        ).wait()

    gather_start(0)
    gather_wait()
    pltpu.make_async_copy(refresh.at[b], bnd, sem_b).wait()

    def chunk(g, carry):
        sbi = (g % NBI) * K
        sbn = ((g + 1) % NBI) * K

        @pl.when(g + 1 < G)
        def _():
            gather_start(g + 1)

        is_avg = jnp.logical_and(g == 0, h == 0)

        @pl.when(is_avg)
        def _():
            for c in range(D // L):
                sl = pl.ds(c * L, L)
                obuf[0, sl] = 0.5 * (ibuf[sbi, sl] + ibuf[sbi + 1, sl])

        @pl.when(jnp.logical_not(is_avg))
        def _():
            _copy_row(obuf, 0, ibuf, sbi + 1)

        def row(i, c2):
            _copy_row(obuf, i, ibuf, sbi + i + 1)
            return c2

        lax.fori_loop(1, K - 1, row, 0)

        @pl.when(g + 1 < G)
        def _():
            gather_wait()
            _copy_row(obuf, K - 1, ibuf, sbn)

        @pl.when(g + 1 == G)
        def _():
            _copy_row(obuf, K - 1, bnd, 0)

        cp = pltpu.make_async_copy(
            obuf, out.at[b, pl.ds(base + g * K, K)], sem_s
        )
        cp.start()
        cp.wait()
        return carry

    lax.fori_loop(0, G, chunk, 0)


@jax.jit
def _shift(bank_states, refresh_states):
    mesh = plsc.VectorSubcoreMesh(core_axis_name="c", subcore_axis_name="s")
    return pl.kernel(
        _body,
        mesh=mesh,
        out_type=jax.ShapeDtypeStruct((B, S, D), jnp.float32),
        scratch_types=[
            pltpu.VMEM((NBI * K, D), jnp.float32),
            pltpu.VMEM((K, D), jnp.float32),
            pltpu.VMEM((1, D), jnp.float32),
            pltpu.SemaphoreType.DMA,
            pltpu.SemaphoreType.DMA,
            pltpu.SemaphoreType.DMA,
        ],
    )(bank_states, refresh_states)


def kernel(bank_states, refresh_states):
    return _shift(bank_states, refresh_states)


# speedup vs baseline: 3.1835x; 2.6284x over previous
import jax
import jax.numpy as jnp
from jax import lax
from jax.experimental import pallas as pl
from jax.experimental.pallas import tpu as pltpu
from jax.experimental.pallas import tpu_sc as plsc

B, S, D = 16, 2048, 1024
N = B * S
L = 16
K = 32
NB = 3
HALF = S // 2
G = HALF // K


def _body(bank, refresh, out, buf, idx, bnd, sem_g, sem_b, sem_s):
    cid = lax.axis_index("c")
    sid = lax.axis_index("s")
    wid = sid * 2 + cid
    b = wid // 2
    h = wid % 2
    wbase = b * S + h * HALF

    @pl.when(h == 0)
    def _():
        pltpu.make_async_copy(bank.at[pl.ds(b * S, 1)], bnd, sem_b).start()

    @pl.when(h == 1)
    def _():
        pltpu.make_async_copy(refresh.at[pl.ds(b, 1)], bnd, sem_b).start()

    def build_idx(g):
        seg = g % NB
        row0 = wbase + g * K + 1
        for c in range(K // L):
            vals = lax.iota(jnp.int32, L) + (row0 + c * L)
            idx[seg, pl.ds(c * L, L)] = jnp.minimum(vals, N - 1)

    def gather_start(g):
        seg = g % NB
        pltpu.make_async_copy(
            bank.at[idx.at[seg]],
            buf.at[pl.ds(seg * K, K)],
            sem_g,
        ).start()

    def gather_wait(g):
        seg = g % NB
        pltpu.make_async_copy(
            bank.at[idx.at[seg]], buf.at[pl.ds(seg * K, K)], sem_g
        ).wait()

    def scatter_start(g):
        seg = g % NB
        pltpu.make_async_copy(
            buf.at[pl.ds(seg * K, K)],
            out.at[pl.ds(wbase + g * K, K)],
            sem_s,
        ).start()

    def scatter_wait():
        pltpu.make_async_copy(
            buf.at[pl.ds(0, K)], out.at[pl.ds(wbase, K)], sem_s
        ).wait()

    build_idx(0)
    gather_start(0)
    pltpu.make_async_copy(refresh.at[pl.ds(b, 1)], bnd, sem_b).wait()

    def chunk(g, carry):
        seg = g % NB

        @pl.when(g >= 2)
        def _():
            scatter_wait()

        @pl.when(g + 1 < G)
        def _():
            build_idx(g + 1)
            gather_start(g + 1)

        gather_wait(g)

        @pl.when(jnp.logical_and(g == 0, h == 0))
        def _():
            for c in range(D // L):
                sl = pl.ds(c * L, L)
                buf[0, sl] = 0.5 * (bnd[0, sl] + buf[0, sl])

        @pl.when(jnp.logical_and(g == G - 1, h == 1))
        def _():
            for c in range(D // L):
                sl = pl.ds(c * L, L)
                buf[seg * K + K - 1, sl] = bnd[0, sl]

        scatter_start(g)
        return carry

    lax.fori_loop(0, G, chunk, 0)
    scatter_wait()
    scatter_wait()


@jax.jit
def _shift(bank2, refresh2):
    mesh = plsc.VectorSubcoreMesh(core_axis_name="c", subcore_axis_name="s")
    return pl.kernel(
        _body,
        mesh=mesh,
        out_type=jax.ShapeDtypeStruct((N, D), jnp.float32),
        scratch_types=[
            pltpu.VMEM((NB * K, D), jnp.float32),
            pltpu.VMEM((NB, K), jnp.int32),
            pltpu.VMEM((1, D), jnp.float32),
            pltpu.SemaphoreType.DMA,
            pltpu.SemaphoreType.DMA,
            pltpu.SemaphoreType.DMA,
        ],
    )(bank2, refresh2)


def kernel(bank_states, refresh_states):
    bank2 = bank_states.reshape(N, D)
    refresh2 = refresh_states.reshape(B, D)
    return _shift(bank2, refresh2).reshape(B, S, D)


# speedup vs baseline: 3.2121x; 1.0090x over previous
import jax
import jax.numpy as jnp
from jax import lax
from jax.experimental import pallas as pl
from jax.experimental.pallas import tpu as pltpu
from jax.experimental.pallas import tpu_sc as plsc

B, S, D = 16, 2048, 1024
N = B * S
L = 16
K = 16
NB = 6
HALF = S // 2
G = HALF // K


def _body(bank, refresh, out, buf, idx, bnd, sem_g, sem_b, sem_s):
    cid = lax.axis_index("c")
    sid = lax.axis_index("s")
    wid = sid * 2 + cid
    b = wid // 2
    h = wid % 2
    wbase = b * S + h * HALF

    @pl.when(h == 0)
    def _():
        pltpu.make_async_copy(bank.at[pl.ds(b * S, 1)], bnd, sem_b).start()

    @pl.when(h == 1)
    def _():
        pltpu.make_async_copy(refresh.at[pl.ds(b, 1)], bnd, sem_b).start()

    def build_idx(g):
        seg = g % NB
        row0 = wbase + g * K + 1
        for c in range(K // L):
            vals = lax.iota(jnp.int32, L) + (row0 + c * L)
            idx[seg, pl.ds(c * L, L)] = jnp.minimum(vals, N - 1)

    def gather_start(g):
        seg = g % NB
        pltpu.make_async_copy(
            bank.at[idx.at[seg]],
            buf.at[pl.ds(seg * K, K)],
            sem_g,
        ).start()

    def gather_wait(g):
        seg = g % NB
        pltpu.make_async_copy(
            bank.at[idx.at[seg]], buf.at[pl.ds(seg * K, K)], sem_g
        ).wait()

    def scatter_start(g):
        seg = g % NB
        pltpu.make_async_copy(
            buf.at[pl.ds(seg * K, K)],
            out.at[pl.ds(wbase + g * K, K)],
            sem_s,
        ).start()

    def scatter_wait():
        pltpu.make_async_copy(
            buf.at[pl.ds(0, K)], out.at[pl.ds(wbase, K)], sem_s
        ).wait()

    build_idx(0)
    gather_start(0)
    build_idx(1)
    gather_start(1)
    pltpu.make_async_copy(refresh.at[pl.ds(b, 1)], bnd, sem_b).wait()

    def chunk(g, carry):
        seg = g % NB

        @pl.when(g + 2 < G)
        def _():
            @pl.when(g >= 4)
            def _():
                scatter_wait()

            build_idx(g + 2)
            gather_start(g + 2)

        gather_wait(g)

        @pl.when(jnp.logical_and(g == 0, h == 0))
        def _():
            for c in range(D // L):
                sl = pl.ds(c * L, L)
                buf[0, sl] = 0.5 * (bnd[0, sl] + buf[0, sl])

        @pl.when(jnp.logical_and(g == G - 1, h == 1))
        def _():
            for c in range(D // L):
                sl = pl.ds(c * L, L)
                buf[seg * K + K - 1, sl] = bnd[0, sl]

        scatter_start(g)
        return carry

    lax.fori_loop(0, G, chunk, 0)
    for _ in range(6):
        scatter_wait()


@jax.jit
def _shift(bank2, refresh2):
    mesh = plsc.VectorSubcoreMesh(core_axis_name="c", subcore_axis_name="s")
    return pl.kernel(
        _body,
        mesh=mesh,
        out_type=jax.ShapeDtypeStruct((N, D), jnp.float32),
        scratch_types=[
            pltpu.VMEM((NB * K, D), jnp.float32),
            pltpu.VMEM((NB, K), jnp.int32),
            pltpu.VMEM((1, D), jnp.float32),
            pltpu.SemaphoreType.DMA,
            pltpu.SemaphoreType.DMA,
            pltpu.SemaphoreType.DMA,
        ],
    )(bank2, refresh2)


def kernel(bank_states, refresh_states):
    bank2 = bank_states.reshape(N, D)
    refresh2 = refresh_states.reshape(B, D)
    return _shift(bank2, refresh2).reshape(B, S, D)
